# initial kernel scaffold (unmeasured)
import functools

import jax
import jax.numpy as jnp
from jax import lax
from jax.experimental import pallas as pl
from jax.experimental.pallas import tpu as pltpu

N_DEV = 32
NEG_INF = -1e9


def kernel(x, Wq, K_ext, V_ext, Wo):
    B, sq, d_model = x.shape
    _, skv, Hq, Dh = K_ext.shape
    d_qk = Wq.shape[1]
    blk = 64
    rows = B * sq
    kv_total = N_DEV * skv

    def body(x_ref, wq_ref, k_ref, v_ref, wo_ref, out_ref,
             kv_ref, ctx_ref, send_sems, recv_sems):
        me = lax.axis_index("i")
        left = lax.rem(me + N_DEV - 1, N_DEV)
        right = lax.rem(me + 1, N_DEV)

        barrier_sem = pltpu.get_barrier_semaphore()
        for nbr in (left, right):
            pl.semaphore_signal(
                barrier_sem, inc=1,
                device_id=(nbr,), device_id_type=pl.DeviceIdType.MESH,
            )
        pl.semaphore_wait(barrier_sem, 2)

        kv_ref[me, 0] = k_ref[...]
        kv_ref[me, 1] = v_ref[...]

        for h in range(1, N_DEV):
            @pl.when((me >= h - 1) & (me < N_DEV - 1))
            def _send():
                o = me - h + 1
                rdma = pltpu.make_async_remote_copy(
                    src_ref=kv_ref.at[o],
                    dst_ref=kv_ref.at[o],
                    send_sem=send_sems.at[h - 1],
                    recv_sem=recv_sems.at[h - 1],
                    device_id=(right,),
                    device_id_type=pl.DeviceIdType.MESH,
                )
                rdma.start()
                rdma.wait_send()

            @pl.when(me >= h)
            def _recv():
                o = me - h
                rdma = pltpu.make_async_remote_copy(
                    src_ref=kv_ref.at[o],
                    dst_ref=kv_ref.at[o],
                    send_sem=send_sems.at[h - 1],
                    recv_sem=recv_sems.at[h - 1],
                    device_id=(left,),
                    device_id_type=pl.DeviceIdType.MESH,
                )
                rdma.wait_recv()

        x2 = x_ref[...].reshape(rows, d_model)
        q2 = jnp.dot(x2, wq_ref[...], preferred_element_type=jnp.float32)
        q2 = q2 * 0.125

        i_blk = me * (sq // blk) + lax.broadcasted_iota(
            jnp.int32, (sq, kv_total), 0) // blk
        j_blk = lax.broadcasted_iota(jnp.int32, (sq, kv_total), 1) // blk
        mask = j_blk <= i_blk

        for b in range(B):
            for hh in range(Hq):
                qbh = q2[b * sq:(b + 1) * sq, hh * Dh:(hh + 1) * Dh]
                kall = kv_ref[:, 0, b, :, hh, :].reshape(kv_total, Dh)
                scores = lax.dot_general(
                    qbh, kall, (((1,), (1,)), ((), ())),
                    preferred_element_type=jnp.float32,
                )
                s = jnp.where(mask, scores, NEG_INF)
                mx = jnp.max(s, axis=1, keepdims=True)
                w = jnp.exp(s - mx)
                w = w / jnp.sum(w, axis=1, keepdims=True)
                vall = kv_ref[:, 1, b, :, hh, :].reshape(kv_total, Dh)
                ctx_ref[b * sq:(b + 1) * sq, hh * Dh:(hh + 1) * Dh] = (
                    lax.dot_general(
                        w, vall, (((1,), (0,)), ((), ())),
                        preferred_element_type=jnp.float32,
                    )
                )

        out2 = jnp.dot(ctx_ref[...], wo_ref[...],
                       preferred_element_type=jnp.float32)
        out_ref[...] = out2.reshape(B, sq, d_model)

    return pl.pallas_call(
        body,
        out_shape=jax.ShapeDtypeStruct((B, sq, d_model), jnp.float32),
        in_specs=[pl.BlockSpec(memory_space=pltpu.VMEM)] * 5,
        out_specs=pl.BlockSpec(memory_space=pltpu.VMEM),
        scratch_shapes=[
            pltpu.VMEM((N_DEV, 2, B, skv, Hq, Dh), jnp.float32),
            pltpu.VMEM((rows, d_qk), jnp.float32),
            pltpu.SemaphoreType.DMA((N_DEV - 1,)),
            pltpu.SemaphoreType.DMA((N_DEV - 1,)),
        ],
        compiler_params=pltpu.CompilerParams(collective_id=0),
    )(x, Wq, K_ext, V_ext, Wo)


# baseline (device time: 83976 ns/iter reference)
import jax
import jax.numpy as jnp
from jax import lax
from jax.experimental import pallas as pl
from jax.experimental.pallas import tpu as pltpu

N_DEV = 32


def kernel(x, Wq, K_ext, V_ext, Wo):
    B, sq, d_model = x.shape
    _, skv, Hq, Dh = K_ext.shape
    d_qk = Wq.shape[1]
    blk = 64
    nblk = sq // blk
    rows = B * sq

    K2 = K_ext.reshape(B, skv, Hq * Dh)
    V2 = V_ext.reshape(B, skv, Hq * Dh)

    def body(x_ref, wq_ref, k_ref, v_ref, wo_ref, out_ref,
             kv_ref, num_ref, den_ref, send_sems, recv_sems):
        me = lax.axis_index("i")
        left = lax.rem(me + N_DEV - 1, N_DEV)
        right = lax.rem(me + 1, N_DEV)

        barrier_sem = pltpu.get_barrier_semaphore()
        for nbr in (left, right):
            pl.semaphore_signal(
                barrier_sem, inc=1,
                device_id=(nbr,), device_id_type=pl.DeviceIdType.MESH,
            )
        pl.semaphore_wait(barrier_sem, 2)

        kv_ref[me, 0] = k_ref[...]
        kv_ref[me, 1] = v_ref[...]

        x2 = x_ref[...].reshape(rows, d_model)
        q2 = jnp.dot(x2, wq_ref[...], preferred_element_type=jnp.float32)
        q2 = q2 * 0.125

        num_ref[...] = jnp.zeros_like(num_ref)
        den_ref[...] = jnp.zeros_like(den_ref)

        r_blk = lax.broadcasted_iota(jnp.int32, (sq, skv), 0) // blk
        c_blk = lax.broadcasted_iota(jnp.int32, (sq, skv), 1) // blk

        def accumulate(o):
            mask = (o * nblk + c_blk) <= (me * nblk + r_blk)
            for b in range(B):
                kb = kv_ref[o, 0, b, :, :]
                vb = kv_ref[o, 1, b, :, :]
                for hh in range(Hq):
                    cs = slice(hh * Dh, (hh + 1) * Dh)
                    qbh = q2[b * sq:(b + 1) * sq, cs]
                    scores = lax.dot_general(
                        qbh, kb[:, cs], (((1,), (1,)), ((), ())),
                        preferred_element_type=jnp.float32,
                    )
                    p = jnp.where(mask, jnp.exp(scores), 0.0)
                    rs = slice(b * sq, (b + 1) * sq)
                    num_ref[rs, cs] += lax.dot_general(
                        p, vb[:, cs], (((1,), (0,)), ((), ())),
                        preferred_element_type=jnp.float32,
                    )
                    den_ref[rs, cs] += jnp.broadcast_to(
                        jnp.sum(p, axis=1, keepdims=True), (sq, Dh))

        for h in range(1, N_DEV):
            send_cond = (me >= h - 1) & (me <= N_DEV - 2)

            def _mk(h=h):
                o = me - h + 1
                return pltpu.make_async_remote_copy(
                    src_ref=kv_ref.at[o],
                    dst_ref=kv_ref.at[o],
                    send_sem=send_sems.at[h - 1],
                    recv_sem=recv_sems.at[h - 1],
                    device_id=(right,),
                    device_id_type=pl.DeviceIdType.MESH,
                )

            @pl.when(send_cond)
            def _start():
                _mk().start()

            if h == 1:
                accumulate(me)
            else:
                @pl.when(me >= h - 1)
                def _acc():
                    accumulate(me - (h - 1))

            @pl.when(send_cond)
            def _drain():
                _mk().wait_send()

            @pl.when(me >= h)
            def _recv():
                o = me - h
                rdma = pltpu.make_async_remote_copy(
                    src_ref=kv_ref.at[o],
                    dst_ref=kv_ref.at[o],
                    send_sem=send_sems.at[h - 1],
                    recv_sem=recv_sems.at[h - 1],
                    device_id=(left,),
                    device_id_type=pl.DeviceIdType.MESH,
                )
                rdma.wait_recv()

        @pl.when(me >= N_DEV - 1)
        def _tail():
            accumulate(me - (N_DEV - 1))

        ctx = num_ref[...] / den_ref[...]
        out2 = jnp.dot(ctx, wo_ref[...], preferred_element_type=jnp.float32)
        out_ref[...] = out2.reshape(B, sq, d_model)

    return pl.pallas_call(
        body,
        out_shape=jax.ShapeDtypeStruct((B, sq, d_model), jnp.float32),
        in_specs=[pl.BlockSpec(memory_space=pltpu.VMEM)] * 5,
        out_specs=pl.BlockSpec(memory_space=pltpu.VMEM),
        scratch_shapes=[
            pltpu.VMEM((N_DEV, 2, B, skv, Hq * Dh), jnp.float32),
            pltpu.VMEM((rows, d_qk), jnp.float32),
            pltpu.VMEM((rows, d_qk), jnp.float32),
            pltpu.SemaphoreType.DMA((N_DEV - 1,)),
            pltpu.SemaphoreType.DMA((N_DEV - 1,)),
        ],
        compiler_params=pltpu.CompilerParams(
            collective_id=0,
            vmem_limit_bytes=56 * 1024 * 1024,
        ),
    )(x, Wq, K2, V2, Wo)


# device time: 63841 ns/iter; 1.3154x vs baseline; 1.3154x over previous
import jax
import jax.numpy as jnp
from jax import lax
from jax.experimental import pallas as pl
from jax.experimental.pallas import tpu as pltpu

N_DEV = 32


def kernel(x, Wq, K_ext, V_ext, Wo):
    B, sq, d_model = x.shape
    _, skv, Hq, Dh = K_ext.shape
    d_qk = Wq.shape[1]
    blk = 64
    nblk = sq // blk
    rows = B * sq

    K2 = K_ext.reshape(B, skv, Hq * Dh).astype(jnp.bfloat16)
    V2 = V_ext.reshape(B, skv, Hq * Dh).astype(jnp.bfloat16)

    def body(x_ref, wq_ref, k_ref, v_ref, wo_ref, out_ref,
             kv_ref, num_ref, den_ref, send_sems, recv_sems):
        me = lax.axis_index("i")
        left = lax.rem(me + N_DEV - 1, N_DEV)
        right = lax.rem(me + 1, N_DEV)

        barrier_sem = pltpu.get_barrier_semaphore()
        for nbr in (left, right):
            pl.semaphore_signal(
                barrier_sem, inc=1,
                device_id=(nbr,), device_id_type=pl.DeviceIdType.MESH,
            )
        pl.semaphore_wait(barrier_sem, 2)

        kv_ref[me, 0] = k_ref[...]
        kv_ref[me, 1] = v_ref[...]

        x2 = x_ref[...].reshape(rows, d_model)
        q2 = jnp.dot(x2, wq_ref[...], preferred_element_type=jnp.float32)
        q16 = (q2 * 0.125).astype(jnp.bfloat16)

        num_ref[...] = jnp.zeros_like(num_ref)
        den_ref[...] = jnp.zeros_like(den_ref)

        r_blk = lax.broadcasted_iota(jnp.int32, (sq, skv), 0) // blk
        c_blk = lax.broadcasted_iota(jnp.int32, (sq, skv), 1) // blk

        def accumulate(o):
            mask = (o * nblk + c_blk) <= (me * nblk + r_blk)
            for b in range(B):
                kb = kv_ref[o, 0, b, :, :]
                vb = kv_ref[o, 1, b, :, :]
                for hh in range(Hq):
                    cs = slice(hh * Dh, (hh + 1) * Dh)
                    qbh = q16[b * sq:(b + 1) * sq, cs]
                    scores = lax.dot_general(
                        qbh, kb[:, cs], (((1,), (1,)), ((), ())),
                        preferred_element_type=jnp.float32,
                    )
                    p = jnp.where(mask, jnp.exp(scores), 0.0)
                    rs = slice(b * sq, (b + 1) * sq)
                    num_ref[rs, cs] += lax.dot_general(
                        p.astype(jnp.bfloat16), vb[:, cs],
                        (((1,), (0,)), ((), ())),
                        preferred_element_type=jnp.float32,
                    )
                    den_ref[rs, cs] += jnp.broadcast_to(
                        jnp.sum(p, axis=1, keepdims=True), (sq, Dh))

        for h in range(1, N_DEV):
            send_cond = (me >= h - 1) & (me <= N_DEV - 2)

            def _mk(h=h):
                o = me - h + 1
                return pltpu.make_async_remote_copy(
                    src_ref=kv_ref.at[o],
                    dst_ref=kv_ref.at[o],
                    send_sem=send_sems.at[h - 1],
                    recv_sem=recv_sems.at[h - 1],
                    device_id=(right,),
                    device_id_type=pl.DeviceIdType.MESH,
                )

            @pl.when(send_cond)
            def _start():
                _mk().start()

            if h == 1:
                accumulate(me)
            else:
                @pl.when(me >= h - 1)
                def _acc():
                    accumulate(me - (h - 1))

            @pl.when(send_cond)
            def _drain():
                _mk().wait_send()

            @pl.when(me >= h)
            def _recv():
                o = me - h
                rdma = pltpu.make_async_remote_copy(
                    src_ref=kv_ref.at[o],
                    dst_ref=kv_ref.at[o],
                    send_sem=send_sems.at[h - 1],
                    recv_sem=recv_sems.at[h - 1],
                    device_id=(left,),
                    device_id_type=pl.DeviceIdType.MESH,
                )
                rdma.wait_recv()

        @pl.when(me >= N_DEV - 1)
        def _tail():
            accumulate(me - (N_DEV - 1))

        ctx = num_ref[...] / den_ref[...]
        out2 = jnp.dot(ctx, wo_ref[...], preferred_element_type=jnp.float32)
        out_ref[...] = out2.reshape(B, sq, d_model)

    return pl.pallas_call(
        body,
        out_shape=jax.ShapeDtypeStruct((B, sq, d_model), jnp.float32),
        in_specs=[pl.BlockSpec(memory_space=pltpu.VMEM)] * 5,
        out_specs=pl.BlockSpec(memory_space=pltpu.VMEM),
        scratch_shapes=[
            pltpu.VMEM((N_DEV, 2, B, skv, Hq * Dh), jnp.bfloat16),
            pltpu.VMEM((rows, d_qk), jnp.float32),
            pltpu.VMEM((rows, d_qk), jnp.float32),
            pltpu.SemaphoreType.DMA((N_DEV - 1,)),
            pltpu.SemaphoreType.DMA((N_DEV - 1,)),
        ],
        compiler_params=pltpu.CompilerParams(
            collective_id=0,
            vmem_limit_bytes=56 * 1024 * 1024,
        ),
    )(x, Wq, K2, V2, Wo)


# device time: 61250 ns/iter; 1.3710x vs baseline; 1.0423x over previous
import jax
import jax.numpy as jnp
from jax import lax
from jax.experimental import pallas as pl
from jax.experimental.pallas import tpu as pltpu

N_DEV = 32


def kernel(x, Wq, K_ext, V_ext, Wo):
    B, sq, d_model = x.shape
    _, skv, Hq, Dh = K_ext.shape
    d_qk = Wq.shape[1]
    blk = 64
    nblk = sq // blk
    rows = B * sq

    K2 = K_ext.reshape(B, skv, Hq * Dh).astype(jnp.bfloat16)
    V2 = V_ext.reshape(B, skv, Hq * Dh).astype(jnp.bfloat16)

    def body(x_ref, wq_ref, k_ref, v_ref, wo_ref, out_ref,
             kv_ref, num_ref, den_ref, send_sems, recv_sems):
        me = lax.axis_index("i")
        left = lax.rem(me + N_DEV - 1, N_DEV)
        right = lax.rem(me + 1, N_DEV)

        barrier_sem = pltpu.get_barrier_semaphore()
        for nbr in (left, right):
            pl.semaphore_signal(
                barrier_sem, inc=1,
                device_id=(nbr,), device_id_type=pl.DeviceIdType.MESH,
            )
        pl.semaphore_wait(barrier_sem, 2)

        kv_ref[me, 0] = k_ref[...]
        kv_ref[me, 1] = v_ref[...]

        x2 = x_ref[...].reshape(rows, d_model)
        q2 = jnp.dot(x2, wq_ref[...], preferred_element_type=jnp.float32)
        q16 = (q2 * 0.125).astype(jnp.bfloat16)

        num_ref[...] = jnp.zeros_like(num_ref)
        den_ref[...] = jnp.zeros_like(den_ref)

        r_blk = lax.broadcasted_iota(jnp.int32, (sq, skv), 0) // blk
        c_blk = lax.broadcasted_iota(jnp.int32, (sq, skv), 1) // blk

        def accumulate(o, masked=True):
            mask = (o * nblk + c_blk) <= (me * nblk + r_blk)
            for b in range(B):
                kb = kv_ref[o, 0, b, :, :]
                vb = kv_ref[o, 1, b, :, :]
                for hh in range(Hq):
                    cs = slice(hh * Dh, (hh + 1) * Dh)
                    qbh = q16[b * sq:(b + 1) * sq, cs]
                    scores = lax.dot_general(
                        qbh, kb[:, cs], (((1,), (1,)), ((), ())),
                        preferred_element_type=jnp.float32,
                    )
                    p = jnp.exp(scores)
                    if masked:
                        p = jnp.where(mask, p, 0.0)
                    rs = slice(b * sq, (b + 1) * sq)
                    num_ref[rs, cs] += lax.dot_general(
                        p.astype(jnp.bfloat16), vb[:, cs],
                        (((1,), (0,)), ((), ())),
                        preferred_element_type=jnp.float32,
                    )
                    den_ref[rs, cs] += jnp.broadcast_to(
                        jnp.sum(p, axis=1, keepdims=True), (sq, Dh))

        for h in range(1, N_DEV):
            send_cond = (me >= h - 1) & (me <= N_DEV - 2)

            def _mk(h=h):
                o = me - h + 1
                return pltpu.make_async_remote_copy(
                    src_ref=kv_ref.at[o],
                    dst_ref=kv_ref.at[o],
                    send_sem=send_sems.at[h - 1],
                    recv_sem=recv_sems.at[h - 1],
                    device_id=(right,),
                    device_id_type=pl.DeviceIdType.MESH,
                )

            @pl.when(send_cond)
            def _start():
                _mk().start()

            if h == 1:
                accumulate(me)
            else:
                @pl.when(me >= h - 1)
                def _acc():
                    accumulate(me - (h - 1), masked=False)

            @pl.when(send_cond)
            def _drain():
                _mk().wait_send()

            @pl.when(me >= h)
            def _recv():
                o = me - h
                rdma = pltpu.make_async_remote_copy(
                    src_ref=kv_ref.at[o],
                    dst_ref=kv_ref.at[o],
                    send_sem=send_sems.at[h - 1],
                    recv_sem=recv_sems.at[h - 1],
                    device_id=(left,),
                    device_id_type=pl.DeviceIdType.MESH,
                )
                rdma.wait_recv()

        @pl.when(me >= N_DEV - 1)
        def _tail():
            accumulate(me - (N_DEV - 1), masked=False)

        ctx = num_ref[...] / den_ref[...]
        out2 = jnp.dot(ctx, wo_ref[...], preferred_element_type=jnp.float32)
        out_ref[...] = out2.reshape(B, sq, d_model)

    return pl.pallas_call(
        body,
        out_shape=jax.ShapeDtypeStruct((B, sq, d_model), jnp.float32),
        in_specs=[pl.BlockSpec(memory_space=pltpu.VMEM)] * 5,
        out_specs=pl.BlockSpec(memory_space=pltpu.VMEM),
        scratch_shapes=[
            pltpu.VMEM((N_DEV, 2, B, skv, Hq * Dh), jnp.bfloat16),
            pltpu.VMEM((rows, d_qk), jnp.float32),
            pltpu.VMEM((rows, d_qk), jnp.float32),
            pltpu.SemaphoreType.DMA((N_DEV - 1,)),
            pltpu.SemaphoreType.DMA((N_DEV - 1,)),
        ],
        compiler_params=pltpu.CompilerParams(
            collective_id=0,
            vmem_limit_bytes=56 * 1024 * 1024,
        ),
    )(x, Wq, K2, V2, Wo)
